# baseline (speedup 1.0000x reference)
import jax
import jax.numpy as jnp
from jax import lax
from jax.experimental import pallas as pl
from jax.experimental.pallas import tpu as pltpu
from jax.experimental.pallas import tpu_sc as plsc

N = 10000
E = 320000
F_IN = 128
H = 64
C = 4
G = 16

NC = 2
NS = 16
NW = NC * NS
EW = E // NW
K = 125
NB = EW // K
N2 = 10240
RPT = N2 // NS
CW = 16

BLK = 1000
M = N // BLK


def _seg_mesh():
    return plsc.VectorSubcoreMesh(
        core_axis_name="c", subcore_axis_name="s", num_cores=NC, num_subcores=NS
    )


def _seg0_body(xa_hbm, xb_hbm, srcr, dstr, zrow, zcnt, ones_hbm,
               outa_hbm, cnt_hbm, outb_hbm,
               srcv, dstv, rba, rbb, onesv, acc, acc_cnt,
               sga, sgb, ssa, ssb, sca, scb):
    c = lax.axis_index("c")
    s = lax.axis_index("s")
    wid = s * NC + c
    rows = pl.ds(s * RPT, RPT)
    pltpu.sync_copy(zrow.at[rows], acc.at[rows])
    pltpu.sync_copy(zcnt.at[rows], acc_cnt.at[rows])
    pltpu.sync_copy(ones_hbm, onesv)
    pltpu.sync_copy(srcr.at[wid], srcv)
    pltpu.sync_copy(dstr.at[wid], dstv)
    plsc.subcore_barrier()

    pltpu.async_copy(xa_hbm.at[srcv.at[0]], rba, sga)
    pltpu.async_copy(xa_hbm.at[srcv.at[1]], rbb, sgb)

    @pl.loop(0, NB // 2)
    def _chunks_a(i):
        j = 2 * i
        pltpu.make_async_copy(xa_hbm.at[srcv.at[0]], rba, sga).wait()
        pltpu.async_copy(rba, acc.at[dstv.at[j]], ssa, add=True)
        pltpu.async_copy(onesv, acc_cnt.at[dstv.at[j]], sca, add=True)
        pltpu.make_async_copy(rba, acc.at[dstv.at[0]], ssa).wait()
        pltpu.make_async_copy(onesv, acc_cnt.at[dstv.at[0]], sca).wait()

        @pl.when(j + 2 < NB)
        def _():
            pltpu.async_copy(xa_hbm.at[srcv.at[j + 2]], rba, sga)

        pltpu.make_async_copy(xa_hbm.at[srcv.at[0]], rbb, sgb).wait()
        pltpu.async_copy(rbb, acc.at[dstv.at[j + 1]], ssb, add=True)
        pltpu.async_copy(onesv, acc_cnt.at[dstv.at[j + 1]], scb, add=True)
        pltpu.make_async_copy(rbb, acc.at[dstv.at[0]], ssb).wait()
        pltpu.make_async_copy(onesv, acc_cnt.at[dstv.at[0]], scb).wait()

        @pl.when(j + 3 < NB)
        def _():
            pltpu.async_copy(xa_hbm.at[srcv.at[j + 3]], rbb, sgb)

    plsc.subcore_barrier()
    pltpu.sync_copy(acc.at[rows], outa_hbm.at[c].at[rows])
    pltpu.sync_copy(acc_cnt.at[rows], cnt_hbm.at[c].at[rows])
    pltpu.sync_copy(zrow.at[rows], acc.at[rows])
    plsc.subcore_barrier()

    pltpu.async_copy(xb_hbm.at[srcv.at[0]], rba, sga)
    pltpu.async_copy(xb_hbm.at[srcv.at[1]], rbb, sgb)

    @pl.loop(0, NB // 2)
    def _chunks_b(i):
        j = 2 * i
        pltpu.make_async_copy(xb_hbm.at[srcv.at[0]], rba, sga).wait()
        pltpu.async_copy(rba, acc.at[dstv.at[j]], ssa, add=True)
        pltpu.make_async_copy(rba, acc.at[dstv.at[0]], ssa).wait()

        @pl.when(j + 2 < NB)
        def _():
            pltpu.async_copy(xb_hbm.at[srcv.at[j + 2]], rba, sga)

        pltpu.make_async_copy(xb_hbm.at[srcv.at[0]], rbb, sgb).wait()
        pltpu.async_copy(rbb, acc.at[dstv.at[j + 1]], ssb, add=True)
        pltpu.make_async_copy(rbb, acc.at[dstv.at[0]], ssb).wait()

        @pl.when(j + 3 < NB)
        def _():
            pltpu.async_copy(xb_hbm.at[srcv.at[j + 3]], rbb, sgb)

    plsc.subcore_barrier()
    pltpu.sync_copy(acc.at[rows], outb_hbm.at[c].at[rows])


def _seg_body(h_hbm, srcr, dstr, zrow, out_hbm,
              srcv, dstv, rba, rbb, acc, sga, sgb, ssa, ssb):
    c = lax.axis_index("c")
    s = lax.axis_index("s")
    wid = s * NC + c
    pltpu.sync_copy(zrow.at[pl.ds(s * RPT, RPT)], acc.at[pl.ds(s * RPT, RPT)])
    pltpu.sync_copy(srcr.at[wid], srcv)
    pltpu.sync_copy(dstr.at[wid], dstv)
    plsc.subcore_barrier()

    pltpu.async_copy(h_hbm.at[srcv.at[0]], rba, sga)
    pltpu.async_copy(h_hbm.at[srcv.at[1]], rbb, sgb)

    @pl.loop(0, NB // 2)
    def _chunks(i):
        j = 2 * i
        pltpu.make_async_copy(h_hbm.at[srcv.at[0]], rba, sga).wait()
        pltpu.async_copy(rba, acc.at[dstv.at[j]], ssa, add=True)
        pltpu.make_async_copy(rba, acc.at[dstv.at[0]], ssa).wait()

        @pl.when(j + 2 < NB)
        def _():
            pltpu.async_copy(h_hbm.at[srcv.at[j + 2]], rba, sga)

        pltpu.make_async_copy(h_hbm.at[srcv.at[0]], rbb, sgb).wait()
        pltpu.async_copy(rbb, acc.at[dstv.at[j + 1]], ssb, add=True)
        pltpu.make_async_copy(rbb, acc.at[dstv.at[0]], ssb).wait()

        @pl.when(j + 3 < NB)
        def _():
            pltpu.async_copy(h_hbm.at[srcv.at[j + 3]], rbb, sgb)

    plsc.subcore_barrier()
    pltpu.sync_copy(acc.at[pl.ds(s * RPT, RPT)],
                    out_hbm.at[c].at[pl.ds(s * RPT, RPT)])


def _seg0(xa, xb, src, dst, zrow, zcnt, ones):
    d = xa.shape[1]
    return pl.kernel(
        _seg0_body,
        out_type=(
            jax.ShapeDtypeStruct((NC, N2, d), jnp.float32),
            jax.ShapeDtypeStruct((NC, N2, CW), jnp.float32),
            jax.ShapeDtypeStruct((NC, N2, d), jnp.float32),
        ),
        mesh=_seg_mesh(),
        compiler_params=pltpu.CompilerParams(use_tc_tiling_on_sc=False),
        scratch_types=[
            pltpu.VMEM((NB, K), jnp.int32),
            pltpu.VMEM((NB, K), jnp.int32),
            pltpu.VMEM((K, d), jnp.float32),
            pltpu.VMEM((K, d), jnp.float32),
            pltpu.VMEM((K, CW), jnp.float32),
            pltpu.VMEM_SHARED((N2, d), jnp.float32),
            pltpu.VMEM_SHARED((N2, CW), jnp.float32),
            pltpu.SemaphoreType.DMA,
            pltpu.SemaphoreType.DMA,
            pltpu.SemaphoreType.DMA,
            pltpu.SemaphoreType.DMA,
            pltpu.SemaphoreType.DMA,
            pltpu.SemaphoreType.DMA,
        ],
    )(xa, xb, src, dst, zrow, zcnt, ones)


def _seg(h, src, dst, zrow):
    d = h.shape[1]
    return pl.kernel(
        _seg_body,
        out_type=jax.ShapeDtypeStruct((NC, N2, d), jnp.float32),
        mesh=_seg_mesh(),
        compiler_params=pltpu.CompilerParams(use_tc_tiling_on_sc=False),
        scratch_types=[
            pltpu.VMEM((NB, K), jnp.int32),
            pltpu.VMEM((NB, K), jnp.int32),
            pltpu.VMEM((K, d), jnp.float32),
            pltpu.VMEM((K, d), jnp.float32),
            pltpu.VMEM_SHARED((N2, d), jnp.float32),
            pltpu.SemaphoreType.DMA,
            pltpu.SemaphoreType.DMA,
            pltpu.SemaphoreType.DMA,
            pltpu.SemaphoreType.DMA,
        ],
    )(h, src, dst, zrow)


def _dotd(a, b):
    return jax.lax.dot_general(a, b, (((1,), (0,)), ((), ())),
                               preferred_element_type=jnp.float32)


def _dot_t_hi(a, b):
    return jax.lax.dot_general(a, b, (((0,), (0,)), ((), ())),
                               preferred_element_type=jnp.float32,
                               precision=jax.lax.Precision.HIGHEST)


def _elu(hn):
    return jnp.where(hn > 0, hn, jnp.exp(hn) - 1.0)


def _block_stats(u, s1_ref, s2_ref):
    bs1 = jnp.sum(u, axis=0, keepdims=True)
    mu_b = bs1 * (1.0 / BLK)
    d = u - mu_b
    s1_ref[...] = bs1
    s2_ref[...] = jnp.sum(d * d, axis=0, keepdims=True)


def _bn_apply(u, s1, s2, g, be):
    mu = jnp.sum(s1, axis=0) * (1.0 / N)
    mu_b = s1 * (1.0 / BLK)
    d = mu_b - mu
    var = (jnp.sum(s2, axis=0)
           + jnp.sum(d * d, axis=0) * BLK) * (1.0 / N)
    return (u - mu) / jnp.sqrt(var + 1e-5) * g + be


def _uh0_body(qpa_ref, qpb_ref, cp_ref, x_ref, wl_ref, bl_ref, wr_ref,
              g_ref, be_ref, h_ref, cm_ref, u_scr, s1_scr, s2_scr):
    b = pl.program_id(0)

    @pl.when(b < M)
    def _():
        cnt = cp_ref[0, :, 0:1] + cp_ref[1, :, 0:1]
        cm = jnp.maximum(cnt, 1.0)
        cm_ref[...] = cm
        mean_a = (qpa_ref[0] + qpa_ref[1]) / cm
        mean_b = (qpb_ref[0] + qpb_ref[1]) / cm
        u = (_dotd(mean_a, wl_ref[0:H, :]) + _dotd(mean_b, wl_ref[H:F_IN, :])
             + bl_ref[...] + _dotd(x_ref[...], wr_ref[...]))
        u_scr[b] = u
        _block_stats(u, s1_scr.at[b], s2_scr.at[b])

    @pl.when(b >= M)
    def _():
        u = u_scr[b - M]
        h_ref[...] = _elu(_bn_apply(u, s1_scr[...], s2_scr[...],
                                    g_ref[...], be_ref[...]))


def _uh_body(qp_ref, cm_ref, hin_ref, wl_ref, bl_ref, wr_ref,
             g_ref, be_ref, h_ref, u_scr, s1_scr, s2_scr):
    b = pl.program_id(0)

    @pl.when(b < M)
    def _():
        mean = (qp_ref[0] + qp_ref[1]) / cm_ref[...]
        u = (_dotd(mean, wl_ref[...]) + bl_ref[...]
             + _dotd(hin_ref[...], wr_ref[...]))
        u_scr[b] = u
        _block_stats(u, s1_scr.at[b], s2_scr.at[b])

    @pl.when(b >= M)
    def _():
        u = u_scr[b - M]
        h_ref[...] = _elu(_bn_apply(u, s1_scr[...], s2_scr[...],
                                    g_ref[...], be_ref[...]))


def _upool_body(qp_ref, cm_ref, hin_ref, wl_ref, bl_ref, wr_ref,
                g_ref, be_ref, batch_ref, w1_ref, b1_ref, gh_ref, bh_ref,
                w2_ref, b2_ref, out_ref,
                u_scr, s1_scr, s2_scr, gsum_ref, gmax_ref, gcnt_ref):
    b = pl.program_id(0)

    @pl.when(b < M)
    def _():
        mean = (qp_ref[0] + qp_ref[1]) / cm_ref[...]
        u = (_dotd(mean, wl_ref[...]) + bl_ref[...]
             + _dotd(hin_ref[...], wr_ref[...]))
        u_scr[b] = u
        _block_stats(u, s1_scr.at[b], s2_scr.at[b])

    @pl.when(b >= M)
    def _():
        h = _elu(_bn_apply(u_scr[b - M], s1_scr[...], s2_scr[...],
                           g_ref[...], be_ref[...]))
        bt = batch_ref[...]
        onehot = (bt == lax.broadcasted_iota(jnp.int32, (1, G), 1)
                  ).astype(jnp.float32)
        psum = _dot_t_hi(onehot, h)
        pcnt = _dot_t_hi(onehot, jnp.ones((BLK, 1), jnp.float32))
        maxs = [jnp.max(jnp.where(bt == gi, h, -jnp.inf), axis=0,
                        keepdims=True) for gi in range(G)]
        pmax = jnp.concatenate(maxs, axis=0)

        @pl.when(b == M)
        def _():
            gsum_ref[...] = psum
            gcnt_ref[...] = pcnt
            gmax_ref[...] = pmax

        @pl.when(b > M)
        def _():
            gsum_ref[...] = gsum_ref[...] + psum
            gcnt_ref[...] = gcnt_ref[...] + pcnt
            gmax_ref[...] = jnp.maximum(gmax_ref[...], pmax)

        @pl.when(b == 2 * M - 1)
        def _():
            mean_pool = gsum_ref[...] / jnp.maximum(gcnt_ref[...], 1.0)
            emb = jnp.concatenate([mean_pool, gmax_ref[...]], axis=1)
            z = _dotd(emb, w1_ref[...]) + b1_ref[...]
            mu = jnp.mean(z, axis=0, keepdims=True)
            var = jnp.mean((z - mu) * (z - mu), axis=0, keepdims=True)
            z = _elu((z - mu) / jnp.sqrt(var + 1e-5) * gh_ref[...]
                     + bh_ref[...])
            out_ref[...] = _dotd(z, w2_ref[...]) + b2_ref[...]


def _full(shape):
    return pl.BlockSpec(shape, lambda b: (0,) * len(shape))


def _rows1(width):
    return pl.BlockSpec((BLK, width), lambda b: (jnp.minimum(b, M - 1), 0))


def _rows2(width):
    return pl.BlockSpec((BLK, width), lambda b: (jnp.maximum(b - M, 0), 0))


def _qp_spec(width):
    return pl.BlockSpec((NC, BLK, width),
                        lambda b: (0, jnp.minimum(b, M - 1), 0))


def _f32(*shape):
    return jax.ShapeDtypeStruct(shape, jnp.float32)


_UH_SCRATCH = [pltpu.VMEM((M, BLK, H), jnp.float32),
               pltpu.VMEM((M, 1, H), jnp.float32),
               pltpu.VMEM((M, 1, H), jnp.float32)]


def _uh0(qpa, qpb, cp, x, Wl, bl, Wr, g, be):
    return pl.pallas_call(
        _uh0_body, grid=(2 * M,),
        in_specs=[_qp_spec(H), _qp_spec(H), _qp_spec(CW), _rows1(F_IN),
                  _full((F_IN, H)), _full((1, H)), _full((F_IN, H)),
                  _full((1, H)), _full((1, H))],
        out_specs=(_rows2(H), _rows1(1)),
        out_shape=(_f32(N, H), _f32(N, 1)),
        scratch_shapes=list(_UH_SCRATCH),
    )(qpa, qpb, cp, x, Wl, bl, Wr, g, be)


def _uh(qp, cm, hin, Wl, bl, Wr, g, be):
    return pl.pallas_call(
        _uh_body, grid=(2 * M,),
        in_specs=[_qp_spec(H), _rows1(1), _rows1(H),
                  _full((H, H)), _full((1, H)), _full((H, H)),
                  _full((1, H)), _full((1, H))],
        out_specs=_rows2(H),
        out_shape=_f32(N, H),
        scratch_shapes=list(_UH_SCRATCH),
    )(qp, cm, hin, Wl, bl, Wr, g, be)


def _upool(qp, cm, hin, Wl, bl, Wr, g, be, batch2,
           W1h, b1h, gh, bh, W2h, b2h):
    return pl.pallas_call(
        _upool_body, grid=(2 * M,),
        in_specs=[_qp_spec(H), _rows1(1), _rows1(H),
                  _full((H, H)), _full((1, H)), _full((H, H)),
                  _full((1, H)), _full((1, H)), _rows2(1),
                  _full((2 * H, H)), _full((1, H)), _full((1, H)),
                  _full((1, H)), _full((H, C)), _full((1, C))],
        out_specs=_full((G, C)),
        out_shape=_f32(G, C),
        scratch_shapes=list(_UH_SCRATCH) + [
            pltpu.VMEM((G, H), jnp.float32),
            pltpu.VMEM((G, H), jnp.float32),
            pltpu.VMEM((G, 1), jnp.float32)],
    )(qp, cm, hin, Wl, bl, Wr, g, be, batch2,
      W1h, b1h, gh, bh, W2h, b2h)


def kernel(x, edge_index, batch, Wl0, bl0, Wr0, g0, be0, Wl1, bl1, Wr1, g1,
           be1, Wl2, bl2, Wr2, g2, be2, W1h, b1h, gh, bh, W2h, b2h):
    src = edge_index[0].reshape(NW, NB, K)
    dst = edge_index[1].reshape(NW, NB, K)
    batch2 = batch.reshape(N, 1)
    zrow = jnp.zeros((N2, H), jnp.float32)
    zcnt = jnp.zeros((N2, CW), jnp.float32)
    ones = jnp.ones((K, CW), jnp.float32)
    r1 = lambda v: v.reshape(1, -1)

    xa = lax.slice(x, (0, 0), (N, H))
    xb = lax.slice(x, (0, H), (N, F_IN))
    q0a, cnt0, q0b = _seg0(xa, xb, src, dst, zrow, zcnt, ones)
    h1, cm = _uh0(q0a, q0b, cnt0, x, Wl0, r1(bl0), Wr0, r1(g0), r1(be0))
    q1 = _seg(h1, src, dst, zrow)
    h2 = _uh(q1, cm, h1, Wl1, r1(bl1), Wr1, r1(g1), r1(be1))
    q2 = _seg(h2, src, dst, zrow)
    return _upool(q2, cm, h2, Wl2, r1(bl2), Wr2, r1(g2), r1(be2), batch2,
                  W1h, r1(b1h), r1(gh), r1(bh), W2h, r1(b2h))


# speedup vs baseline: 1.0126x; 1.0126x over previous
import jax
import jax.numpy as jnp
from jax import lax
from jax.experimental import pallas as pl
from jax.experimental.pallas import tpu as pltpu
from jax.experimental.pallas import tpu_sc as plsc

N = 10000
E = 320000
F_IN = 128
H = 64
C = 4
G = 16

NC = 2
NS = 16
NW = NC * NS
EW = E // NW
K = 125
NB = EW // K
N2 = 10240
RPT = N2 // NS
CW = 16

BLK = 1000
M = N // BLK


def _seg_mesh():
    return plsc.VectorSubcoreMesh(
        core_axis_name="c", subcore_axis_name="s", num_cores=NC, num_subcores=NS
    )


def _seg0_body(h_hbm, srcr, dstr, zrow, zcnt, ones_hbm, out_hbm, cnt_hbm,
               srcv, dstv, rba, rbb, onesv, acc, acc_cnt,
               sga, sgb, ssa, ssb, sca, scb):
    c = lax.axis_index("c")
    s = lax.axis_index("s")
    wid = s * NC + c
    rows = pl.ds(s * RPT, RPT)
    pltpu.sync_copy(zrow.at[rows], acc.at[rows])
    pltpu.sync_copy(zcnt.at[rows], acc_cnt.at[rows])
    pltpu.sync_copy(ones_hbm, onesv)
    pltpu.sync_copy(srcr.at[wid], srcv)
    pltpu.sync_copy(dstr.at[wid], dstv)
    plsc.subcore_barrier()

    pltpu.async_copy(h_hbm.at[srcv.at[0]], rba, sga)
    pltpu.async_copy(h_hbm.at[srcv.at[1]], rbb, sgb)

    @pl.loop(0, NB // 2)
    def _chunks(i):
        j = 2 * i
        pltpu.make_async_copy(h_hbm.at[srcv.at[0]], rba, sga).wait()
        pltpu.async_copy(rba, acc.at[dstv.at[j]], ssa, add=True)
        pltpu.async_copy(onesv, acc_cnt.at[dstv.at[j]], sca, add=True)
        pltpu.make_async_copy(rba, acc.at[dstv.at[0]], ssa).wait()
        pltpu.make_async_copy(onesv, acc_cnt.at[dstv.at[0]], sca).wait()

        @pl.when(j + 2 < NB)
        def _():
            pltpu.async_copy(h_hbm.at[srcv.at[j + 2]], rba, sga)

        pltpu.make_async_copy(h_hbm.at[srcv.at[0]], rbb, sgb).wait()
        pltpu.async_copy(rbb, acc.at[dstv.at[j + 1]], ssb, add=True)
        pltpu.async_copy(onesv, acc_cnt.at[dstv.at[j + 1]], scb, add=True)
        pltpu.make_async_copy(rbb, acc.at[dstv.at[0]], ssb).wait()
        pltpu.make_async_copy(onesv, acc_cnt.at[dstv.at[0]], scb).wait()

        @pl.when(j + 3 < NB)
        def _():
            pltpu.async_copy(h_hbm.at[srcv.at[j + 3]], rbb, sgb)

    plsc.subcore_barrier()
    pltpu.sync_copy(acc.at[rows], out_hbm.at[c].at[rows])
    pltpu.sync_copy(acc_cnt.at[rows], cnt_hbm.at[c].at[rows])


def _seg_body(h_hbm, srcr, dstr, zrow, out_hbm,
              srcv, dstv, rba, rbb, acc, sga, sgb, ssa, ssb):
    c = lax.axis_index("c")
    s = lax.axis_index("s")
    wid = s * NC + c
    pltpu.sync_copy(zrow.at[pl.ds(s * RPT, RPT)], acc.at[pl.ds(s * RPT, RPT)])
    pltpu.sync_copy(srcr.at[wid], srcv)
    pltpu.sync_copy(dstr.at[wid], dstv)
    plsc.subcore_barrier()

    pltpu.async_copy(h_hbm.at[srcv.at[0]], rba, sga)
    pltpu.async_copy(h_hbm.at[srcv.at[1]], rbb, sgb)

    @pl.loop(0, NB // 2)
    def _chunks(i):
        j = 2 * i
        pltpu.make_async_copy(h_hbm.at[srcv.at[0]], rba, sga).wait()
        pltpu.async_copy(rba, acc.at[dstv.at[j]], ssa, add=True)
        pltpu.make_async_copy(rba, acc.at[dstv.at[0]], ssa).wait()

        @pl.when(j + 2 < NB)
        def _():
            pltpu.async_copy(h_hbm.at[srcv.at[j + 2]], rba, sga)

        pltpu.make_async_copy(h_hbm.at[srcv.at[0]], rbb, sgb).wait()
        pltpu.async_copy(rbb, acc.at[dstv.at[j + 1]], ssb, add=True)
        pltpu.make_async_copy(rbb, acc.at[dstv.at[0]], ssb).wait()

        @pl.when(j + 3 < NB)
        def _():
            pltpu.async_copy(h_hbm.at[srcv.at[j + 3]], rbb, sgb)

    plsc.subcore_barrier()
    pltpu.sync_copy(acc.at[pl.ds(s * RPT, RPT)],
                    out_hbm.at[c].at[pl.ds(s * RPT, RPT)])


def _seg0(h, src, dst, zrow, zcnt, ones):
    d = h.shape[1]
    return pl.kernel(
        _seg0_body,
        out_type=(
            jax.ShapeDtypeStruct((NC, N2, d), jnp.float32),
            jax.ShapeDtypeStruct((NC, N2, CW), jnp.float32),
        ),
        mesh=_seg_mesh(),
        compiler_params=pltpu.CompilerParams(use_tc_tiling_on_sc=False),
        scratch_types=[
            pltpu.VMEM((NB, K), jnp.int32),
            pltpu.VMEM((NB, K), jnp.int32),
            pltpu.VMEM((K, d), jnp.float32),
            pltpu.VMEM((K, d), jnp.float32),
            pltpu.VMEM((K, CW), jnp.float32),
            pltpu.VMEM_SHARED((N2, d), jnp.float32),
            pltpu.VMEM_SHARED((N2, CW), jnp.float32),
            pltpu.SemaphoreType.DMA,
            pltpu.SemaphoreType.DMA,
            pltpu.SemaphoreType.DMA,
            pltpu.SemaphoreType.DMA,
            pltpu.SemaphoreType.DMA,
            pltpu.SemaphoreType.DMA,
        ],
    )(h, src, dst, zrow, zcnt, ones)


def _seg(h, src, dst, zrow):
    d = h.shape[1]
    return pl.kernel(
        _seg_body,
        out_type=jax.ShapeDtypeStruct((NC, N2, d), jnp.float32),
        mesh=_seg_mesh(),
        compiler_params=pltpu.CompilerParams(use_tc_tiling_on_sc=False),
        scratch_types=[
            pltpu.VMEM((NB, K), jnp.int32),
            pltpu.VMEM((NB, K), jnp.int32),
            pltpu.VMEM((K, d), jnp.float32),
            pltpu.VMEM((K, d), jnp.float32),
            pltpu.VMEM_SHARED((N2, d), jnp.float32),
            pltpu.SemaphoreType.DMA,
            pltpu.SemaphoreType.DMA,
            pltpu.SemaphoreType.DMA,
            pltpu.SemaphoreType.DMA,
        ],
    )(h, src, dst, zrow)


def _dotd(a, b):
    return jax.lax.dot_general(a, b, (((1,), (0,)), ((), ())),
                               preferred_element_type=jnp.float32)


def _dot_t_hi(a, b):
    return jax.lax.dot_general(a, b, (((0,), (0,)), ((), ())),
                               preferred_element_type=jnp.float32,
                               precision=jax.lax.Precision.HIGHEST)


def _elu(hn):
    return jnp.where(hn > 0, hn, jnp.exp(hn) - 1.0)


def _block_stats(u, s1_ref, s2_ref):
    bs1 = jnp.sum(u, axis=0, keepdims=True)
    mu_b = bs1 * (1.0 / BLK)
    d = u - mu_b
    s1_ref[...] = bs1
    s2_ref[...] = jnp.sum(d * d, axis=0, keepdims=True)


def _bn_apply(u, s1, s2, g, be):
    mu = jnp.sum(s1, axis=0) * (1.0 / N)
    mu_b = s1 * (1.0 / BLK)
    d = mu_b - mu
    var = (jnp.sum(s2, axis=0)
           + jnp.sum(d * d, axis=0) * BLK) * (1.0 / N)
    return (u - mu) / jnp.sqrt(var + 1e-5) * g + be


def _uh0_body(qpa_ref, qpb_ref, cp_ref, x_ref, wl_ref, bl_ref, wr_ref,
              g_ref, be_ref, h_ref, cm_ref, u_scr, s1_scr, s2_scr):
    b = pl.program_id(0)

    @pl.when(b < M)
    def _():
        cnt = cp_ref[0, :, 0:1] + cp_ref[1, :, 0:1]
        cm = jnp.maximum(cnt, 1.0)
        cm_ref[...] = cm
        mean_a = (qpa_ref[0] + qpa_ref[1]) / cm
        mean_b = (qpb_ref[0] + qpb_ref[1]) / cm
        u = (_dotd(mean_a, wl_ref[0:H, :]) + _dotd(mean_b, wl_ref[H:F_IN, :])
             + bl_ref[...] + _dotd(x_ref[...], wr_ref[...]))
        u_scr[b] = u
        _block_stats(u, s1_scr.at[b], s2_scr.at[b])

    @pl.when(b >= M)
    def _():
        u = u_scr[b - M]
        h_ref[...] = _elu(_bn_apply(u, s1_scr[...], s2_scr[...],
                                    g_ref[...], be_ref[...]))


def _uh_body(qp_ref, cm_ref, hin_ref, wl_ref, bl_ref, wr_ref,
             g_ref, be_ref, h_ref, u_scr, s1_scr, s2_scr):
    b = pl.program_id(0)

    @pl.when(b < M)
    def _():
        mean = (qp_ref[0] + qp_ref[1]) / cm_ref[...]
        u = (_dotd(mean, wl_ref[...]) + bl_ref[...]
             + _dotd(hin_ref[...], wr_ref[...]))
        u_scr[b] = u
        _block_stats(u, s1_scr.at[b], s2_scr.at[b])

    @pl.when(b >= M)
    def _():
        u = u_scr[b - M]
        h_ref[...] = _elu(_bn_apply(u, s1_scr[...], s2_scr[...],
                                    g_ref[...], be_ref[...]))


def _upool_body(qp_ref, cm_ref, hin_ref, wl_ref, bl_ref, wr_ref,
                g_ref, be_ref, batch_ref, w1_ref, b1_ref, gh_ref, bh_ref,
                w2_ref, b2_ref, out_ref,
                u_scr, s1_scr, s2_scr, gsum_ref, gmax_ref, gcnt_ref):
    b = pl.program_id(0)

    @pl.when(b < M)
    def _():
        mean = (qp_ref[0] + qp_ref[1]) / cm_ref[...]
        u = (_dotd(mean, wl_ref[...]) + bl_ref[...]
             + _dotd(hin_ref[...], wr_ref[...]))
        u_scr[b] = u
        _block_stats(u, s1_scr.at[b], s2_scr.at[b])

    @pl.when(b >= M)
    def _():
        h = _elu(_bn_apply(u_scr[b - M], s1_scr[...], s2_scr[...],
                           g_ref[...], be_ref[...]))
        bt = batch_ref[...]
        onehot = (bt == lax.broadcasted_iota(jnp.int32, (1, G), 1)
                  ).astype(jnp.float32)
        psum = _dot_t_hi(onehot, h)
        pcnt = _dot_t_hi(onehot, jnp.ones((BLK, 1), jnp.float32))
        maxs = [jnp.max(jnp.where(bt == gi, h, -jnp.inf), axis=0,
                        keepdims=True) for gi in range(G)]
        pmax = jnp.concatenate(maxs, axis=0)

        @pl.when(b == M)
        def _():
            gsum_ref[...] = psum
            gcnt_ref[...] = pcnt
            gmax_ref[...] = pmax

        @pl.when(b > M)
        def _():
            gsum_ref[...] = gsum_ref[...] + psum
            gcnt_ref[...] = gcnt_ref[...] + pcnt
            gmax_ref[...] = jnp.maximum(gmax_ref[...], pmax)

        @pl.when(b == 2 * M - 1)
        def _():
            mean_pool = gsum_ref[...] / jnp.maximum(gcnt_ref[...], 1.0)
            emb = jnp.concatenate([mean_pool, gmax_ref[...]], axis=1)
            z = _dotd(emb, w1_ref[...]) + b1_ref[...]
            mu = jnp.mean(z, axis=0, keepdims=True)
            var = jnp.mean((z - mu) * (z - mu), axis=0, keepdims=True)
            z = _elu((z - mu) / jnp.sqrt(var + 1e-5) * gh_ref[...]
                     + bh_ref[...])
            out_ref[...] = _dotd(z, w2_ref[...]) + b2_ref[...]


def _full(shape):
    return pl.BlockSpec(shape, lambda b: (0,) * len(shape))


def _rows1(width):
    return pl.BlockSpec((BLK, width), lambda b: (jnp.minimum(b, M - 1), 0))


def _rows2(width):
    return pl.BlockSpec((BLK, width), lambda b: (jnp.maximum(b - M, 0), 0))


def _qp_spec(width):
    return pl.BlockSpec((NC, BLK, width),
                        lambda b: (0, jnp.minimum(b, M - 1), 0))


def _f32(*shape):
    return jax.ShapeDtypeStruct(shape, jnp.float32)


_UH_SCRATCH = [pltpu.VMEM((M, BLK, H), jnp.float32),
               pltpu.VMEM((M, 1, H), jnp.float32),
               pltpu.VMEM((M, 1, H), jnp.float32)]


def _uh0(qpa, qpb, cp, x, Wl, bl, Wr, g, be):
    return pl.pallas_call(
        _uh0_body, grid=(2 * M,),
        in_specs=[_qp_spec(H), _qp_spec(H), _qp_spec(CW), _rows1(F_IN),
                  _full((F_IN, H)), _full((1, H)), _full((F_IN, H)),
                  _full((1, H)), _full((1, H))],
        out_specs=(_rows2(H), _rows1(1)),
        out_shape=(_f32(N, H), _f32(N, 1)),
        scratch_shapes=list(_UH_SCRATCH),
    )(qpa, qpb, cp, x, Wl, bl, Wr, g, be)


def _uh(qp, cm, hin, Wl, bl, Wr, g, be):
    return pl.pallas_call(
        _uh_body, grid=(2 * M,),
        in_specs=[_qp_spec(H), _rows1(1), _rows1(H),
                  _full((H, H)), _full((1, H)), _full((H, H)),
                  _full((1, H)), _full((1, H))],
        out_specs=_rows2(H),
        out_shape=_f32(N, H),
        scratch_shapes=list(_UH_SCRATCH),
    )(qp, cm, hin, Wl, bl, Wr, g, be)


def _upool(qp, cm, hin, Wl, bl, Wr, g, be, batch2,
           W1h, b1h, gh, bh, W2h, b2h):
    return pl.pallas_call(
        _upool_body, grid=(2 * M,),
        in_specs=[_qp_spec(H), _rows1(1), _rows1(H),
                  _full((H, H)), _full((1, H)), _full((H, H)),
                  _full((1, H)), _full((1, H)), _rows2(1),
                  _full((2 * H, H)), _full((1, H)), _full((1, H)),
                  _full((1, H)), _full((H, C)), _full((1, C))],
        out_specs=_full((G, C)),
        out_shape=_f32(G, C),
        scratch_shapes=list(_UH_SCRATCH) + [
            pltpu.VMEM((G, H), jnp.float32),
            pltpu.VMEM((G, H), jnp.float32),
            pltpu.VMEM((G, 1), jnp.float32)],
    )(qp, cm, hin, Wl, bl, Wr, g, be, batch2,
      W1h, b1h, gh, bh, W2h, b2h)


def kernel(x, edge_index, batch, Wl0, bl0, Wr0, g0, be0, Wl1, bl1, Wr1, g1,
           be1, Wl2, bl2, Wr2, g2, be2, W1h, b1h, gh, bh, W2h, b2h):
    src = edge_index[0].reshape(NW, NB, K)
    dst = edge_index[1].reshape(NW, NB, K)
    batch2 = batch.reshape(N, 1)
    zrow = jnp.zeros((N2, H), jnp.float32)
    zcnt = jnp.zeros((N2, CW), jnp.float32)
    ones = jnp.ones((K, CW), jnp.float32)
    r1 = lambda v: v.reshape(1, -1)

    xa = lax.slice(x, (0, 0), (N, H))
    xb = lax.slice(x, (0, H), (N, F_IN))
    q0a, cnt0 = _seg0(xa, src, dst, zrow, zcnt, ones)
    q0b = _seg(xb, src, dst, zrow)
    h1, cm = _uh0(q0a, q0b, cnt0, x, Wl0, r1(bl0), Wr0, r1(g0), r1(be0))
    q1 = _seg(h1, src, dst, zrow)
    h2 = _uh(q1, cm, h1, Wl1, r1(bl1), Wr1, r1(g1), r1(be1))
    q2 = _seg(h2, src, dst, zrow)
    return _upool(q2, cm, h2, Wl2, r1(bl2), Wr2, r1(g2), r1(be2), batch2,
                  W1h, r1(b1h), r1(gh), r1(bh), W2h, r1(b2h))
